# baseline (speedup 1.0000x reference)
import functools

import jax
import jax.numpy as jnp
from jax import lax
from jax.experimental import pallas as pl
from jax.experimental.pallas import tpu as pltpu
from jax.experimental.pallas import tpu_sc as plsc

B = 16384
D = 128
H1 = 256
H2 = 128

NW = 32
CHUNK = 128
IDX_COLS = 128
NSPLIT = 4
HALF = B // NSPLIT


def _gather_body(part, h2, r2, t2, ent, rel, out_h, out_r, out_t,
                 hidx, ridx, tidx, buf0, buf1, buf2, buf3,
                 gsem0, gsem1, wsem0, wsem1, wsem2, wsem3):
    rows_per_w = HALF // NW
    chunks_per_w = max(1, rows_per_w // CHUNK)
    c = lax.axis_index("c")
    s = lax.axis_index("s")
    wid = s * 2 + c
    row0 = part * (HALF // IDX_COLS) + wid * chunks_per_w
    base = wid * rows_per_w

    pltpu.sync_copy(h2.at[pl.ds(row0, chunks_per_w)], hidx)
    pltpu.sync_copy(r2.at[pl.ds(row0, chunks_per_w)], ridx)
    pltpu.sync_copy(t2.at[pl.ds(row0, chunks_per_w)], tidx)

    jobs = []
    for tbl, idx, out in ((ent, hidx, out_h), (rel, ridx, out_r),
                          (ent, tidx, out_t)):
        for j in range(chunks_per_w):
            jobs.append((tbl, idx, j, out))
    n = len(jobs)
    bufs = (buf0, buf1, buf2, buf3)
    gsems = (gsem0, gsem1)
    wsems = (wsem0, wsem1, wsem2, wsem3)

    def start_gather(k):
        tbl, idx, j, _ = jobs[k]
        return pltpu.async_copy(tbl.at[idx.at[j]], bufs[k % 4], gsems[k % 2])

    def start_wb(k):
        tbl, idx, j, out = jobs[k]
        return pltpu.async_copy(bufs[k % 4], out.at[pl.ds(base + j * CHUNK, CHUNK)],
                                wsems[k % 4])

    g = [None] * n
    wb = [None] * n
    g[0] = start_gather(0)
    if n > 1:
        g[1] = start_gather(1)
    for k in range(n):
        g[k].wait()
        wb[k] = start_wb(k)
        if k + 2 < n:
            if k - 2 >= 0:
                wb[k - 2].wait()
            g[k + 2] = start_gather(k + 2)
    for k in range(max(0, n - 4), n):
        wb[k].wait()


@functools.cache
def _gather_call(part):
    return functools.partial(
        pl.kernel,
        mesh=plsc.VectorSubcoreMesh(core_axis_name="c", subcore_axis_name="s"),
        out_type=[
            jax.ShapeDtypeStruct((HALF, D), jnp.float32),
            jax.ShapeDtypeStruct((HALF, D), jnp.float32),
            jax.ShapeDtypeStruct((HALF, D), jnp.float32),
        ],
        scratch_types=[
            pltpu.VMEM((HALF // NW // CHUNK, IDX_COLS), jnp.int32),
            pltpu.VMEM((HALF // NW // CHUNK, IDX_COLS), jnp.int32),
            pltpu.VMEM((HALF // NW // CHUNK, IDX_COLS), jnp.int32),
            pltpu.VMEM((CHUNK, D), jnp.float32),
            pltpu.VMEM((CHUNK, D), jnp.float32),
            pltpu.VMEM((CHUNK, D), jnp.float32),
            pltpu.VMEM((CHUNK, D), jnp.float32),
            pltpu.SemaphoreType.DMA,
            pltpu.SemaphoreType.DMA,
            pltpu.SemaphoreType.DMA,
            pltpu.SemaphoreType.DMA,
            pltpu.SemaphoreType.DMA,
            pltpu.SemaphoreType.DMA,
        ],
    )(functools.partial(_gather_body, part))


BLK = 2048


def _mlp_body(h_ref, r_ref, t_ref, w1h, w1r, w1t, b1, w2, b2, w3r, b3, out_ref):
    x = (jnp.dot(h_ref[...], w1h[...], preferred_element_type=jnp.float32)
         + jnp.dot(r_ref[...], w1r[...], preferred_element_type=jnp.float32)
         + jnp.dot(t_ref[...], w1t[...], preferred_element_type=jnp.float32)
         + b1[...])
    x = jnp.maximum(x, 0.0)
    x = jnp.dot(x, w2[...], preferred_element_type=jnp.float32) + b2[...]
    x = jnp.maximum(x, 0.0)
    o = lax.dot_general(w3r[...], x, (((1,), (1,)), ((), ())),
                        preferred_element_type=jnp.float32) + b3[0]
    out_ref[...] = jax.nn.sigmoid(o).reshape(1, 1, BLK)


def _mlp_body_aliased(h_ref, r_ref, t_ref, w1h, w1r, w1t, b1, w2, b2, w3r, b3,
                      oprev, out_ref):
    del oprev
    _mlp_body(h_ref, r_ref, t_ref, w1h, w1r, w1t, b1, w2, b2, w3r, b3, out_ref)


def _mlp_part(part, h_e, r_e, t_e, w1h, w1r, w1t, b1_2, W2, b2_2, W3, b3,
              o_prev=None):
    grid = (HALF // BLK,)
    blk0 = part * (HALF // BLK)
    full = lambda i: (0, 0)
    in_specs = [
        pl.BlockSpec((BLK, D), lambda i: (i, 0)),
        pl.BlockSpec((BLK, D), lambda i: (i, 0)),
        pl.BlockSpec((BLK, D), lambda i: (i, 0)),
        pl.BlockSpec((D, H1), full),
        pl.BlockSpec((D, H1), full),
        pl.BlockSpec((D, H1), full),
        pl.BlockSpec((1, H1), full),
        pl.BlockSpec((H1, H2), full),
        pl.BlockSpec((1, H2), full),
        pl.BlockSpec((1, H2), full),
        pl.BlockSpec(memory_space=pltpu.SMEM),
    ]
    args = [h_e, r_e, t_e, w1h, w1r, w1t, b1_2, W2, b2_2, W3, b3]
    body = _mlp_body
    aliases = {}
    if o_prev is not None:
        in_specs.append(pl.BlockSpec(memory_space=pl.ANY))
        args.append(o_prev)
        body = _mlp_body_aliased
        aliases = {11: 0}
    return pl.pallas_call(
        body,
        grid=grid,
        in_specs=in_specs,
        out_specs=pl.BlockSpec((1, 1, BLK), lambda i, blk0=blk0: (i + blk0, 0, 0)),
        out_shape=jax.ShapeDtypeStruct((B // BLK, 1, BLK), jnp.float32),
        input_output_aliases=aliases,
    )(*args)


@jax.jit
def kernel(h, r, t, entity_table, relation_table, W1, b1, W2, b2, W3, b3):
    h2 = h.reshape(B // IDX_COLS, IDX_COLS)
    r2 = r.reshape(B // IDX_COLS, IDX_COLS)
    t2 = t.reshape(B // IDX_COLS, IDX_COLS)
    w1h = W1[0:D]
    w1r = W1[D:2 * D]
    w1t = W1[2 * D:3 * D]
    b1_2 = b1.reshape(1, H1)
    b2_2 = b2.reshape(1, H2)
    w3_row = W3.reshape(1, H2)

    o = None
    for p in range(NSPLIT):
        e = _gather_call(p)(h2, r2, t2, entity_table, relation_table)
        o = _mlp_part(p, *e, w1h, w1r, w1t, b1_2, W2, b2_2, w3_row, b3,
                      o_prev=o)
    return o.reshape(B, 1)


# speedup vs baseline: 1.1600x; 1.1600x over previous
import functools

import jax
import jax.numpy as jnp
from jax import lax
from jax.experimental import pallas as pl
from jax.experimental.pallas import tpu as pltpu
from jax.experimental.pallas import tpu_sc as plsc

B = 16384
D = 128
H1 = 256
H2 = 128

NW = 32
CHUNK = 128
IDX_COLS = 128
NSPLIT = 2
HALF = B // NSPLIT


def _gather_body(part, h2, r2, t2, ent, rel, out_h, out_r, out_t,
                 hidx, ridx, tidx,
                 buf0, buf1, buf2, buf3, buf4, buf5,
                 gsem0, gsem1, gsem2, gsem3, gsem4, gsem5,
                 wsem0, wsem1, wsem2, wsem3, wsem4, wsem5):
    rows_per_w = HALF // NW
    chunks_per_w = max(1, rows_per_w // CHUNK)
    c = lax.axis_index("c")
    s = lax.axis_index("s")
    wid = s * 2 + c
    row0 = part * (HALF // IDX_COLS) + wid * chunks_per_w
    base = wid * rows_per_w

    pltpu.sync_copy(h2.at[pl.ds(row0, chunks_per_w)], hidx)
    pltpu.sync_copy(r2.at[pl.ds(row0, chunks_per_w)], ridx)
    pltpu.sync_copy(t2.at[pl.ds(row0, chunks_per_w)], tidx)

    jobs = []
    for tbl, idx, out in ((ent, hidx, out_h), (rel, ridx, out_r),
                          (ent, tidx, out_t)):
        for j in range(chunks_per_w):
            jobs.append((tbl, idx, j, out))
    n = len(jobs)
    bufs = (buf0, buf1, buf2, buf3, buf4, buf5)
    gsems = (gsem0, gsem1, gsem2, gsem3, gsem4, gsem5)
    wsems = (wsem0, wsem1, wsem2, wsem3, wsem4, wsem5)

    g = [pltpu.async_copy(jobs[k][0].at[jobs[k][1].at[jobs[k][2]]],
                          bufs[k], gsems[k]) for k in range(n)]
    wb = [None] * n
    for k in range(n):
        tbl, idx, j, out = jobs[k]
        g[k].wait()
        wb[k] = pltpu.async_copy(bufs[k],
                                 out.at[pl.ds(base + j * CHUNK, CHUNK)],
                                 wsems[k])
    for k in range(n):
        wb[k].wait()


@functools.cache
def _gather_call(part):
    return functools.partial(
        pl.kernel,
        mesh=plsc.VectorSubcoreMesh(core_axis_name="c", subcore_axis_name="s"),
        out_type=[
            jax.ShapeDtypeStruct((HALF, D), jnp.float32),
            jax.ShapeDtypeStruct((HALF, D), jnp.float32),
            jax.ShapeDtypeStruct((HALF, D), jnp.float32),
        ],
        scratch_types=[
            pltpu.VMEM((HALF // NW // CHUNK, IDX_COLS), jnp.int32),
            pltpu.VMEM((HALF // NW // CHUNK, IDX_COLS), jnp.int32),
            pltpu.VMEM((HALF // NW // CHUNK, IDX_COLS), jnp.int32),
            pltpu.VMEM((CHUNK, D), jnp.float32),
            pltpu.VMEM((CHUNK, D), jnp.float32),
            pltpu.VMEM((CHUNK, D), jnp.float32),
            pltpu.VMEM((CHUNK, D), jnp.float32),
            pltpu.VMEM((CHUNK, D), jnp.float32),
            pltpu.VMEM((CHUNK, D), jnp.float32),
            pltpu.SemaphoreType.DMA,
            pltpu.SemaphoreType.DMA,
            pltpu.SemaphoreType.DMA,
            pltpu.SemaphoreType.DMA,
            pltpu.SemaphoreType.DMA,
            pltpu.SemaphoreType.DMA,
            pltpu.SemaphoreType.DMA,
            pltpu.SemaphoreType.DMA,
            pltpu.SemaphoreType.DMA,
            pltpu.SemaphoreType.DMA,
            pltpu.SemaphoreType.DMA,
            pltpu.SemaphoreType.DMA,
        ],
    )(functools.partial(_gather_body, part))


BLK = 2048


def _mlp_body(h_ref, r_ref, t_ref, w1h, w1r, w1t, b1, w2, b2, w3r, b3, out_ref):
    x = (jnp.dot(h_ref[...], w1h[...], preferred_element_type=jnp.float32)
         + jnp.dot(r_ref[...], w1r[...], preferred_element_type=jnp.float32)
         + jnp.dot(t_ref[...], w1t[...], preferred_element_type=jnp.float32)
         + b1[...])
    x = jnp.maximum(x, 0.0)
    x = jnp.dot(x, w2[...], preferred_element_type=jnp.float32) + b2[...]
    x = jnp.maximum(x, 0.0)
    o = lax.dot_general(w3r[...], x, (((1,), (1,)), ((), ())),
                        preferred_element_type=jnp.float32) + b3[0]
    out_ref[...] = jax.nn.sigmoid(o).reshape(1, 1, BLK)


def _mlp_body_aliased(h_ref, r_ref, t_ref, w1h, w1r, w1t, b1, w2, b2, w3r, b3,
                      oprev, out_ref):
    del oprev
    _mlp_body(h_ref, r_ref, t_ref, w1h, w1r, w1t, b1, w2, b2, w3r, b3, out_ref)


def _mlp_part(part, h_e, r_e, t_e, w1h, w1r, w1t, b1_2, W2, b2_2, W3, b3,
              o_prev=None):
    grid = (HALF // BLK,)
    blk0 = part * (HALF // BLK)
    full = lambda i: (0, 0)
    in_specs = [
        pl.BlockSpec((BLK, D), lambda i: (i, 0)),
        pl.BlockSpec((BLK, D), lambda i: (i, 0)),
        pl.BlockSpec((BLK, D), lambda i: (i, 0)),
        pl.BlockSpec((D, H1), full),
        pl.BlockSpec((D, H1), full),
        pl.BlockSpec((D, H1), full),
        pl.BlockSpec((1, H1), full),
        pl.BlockSpec((H1, H2), full),
        pl.BlockSpec((1, H2), full),
        pl.BlockSpec((1, H2), full),
        pl.BlockSpec(memory_space=pltpu.SMEM),
    ]
    args = [h_e, r_e, t_e, w1h, w1r, w1t, b1_2, W2, b2_2, W3, b3]
    body = _mlp_body
    aliases = {}
    if o_prev is not None:
        in_specs.append(pl.BlockSpec(memory_space=pl.ANY))
        args.append(o_prev)
        body = _mlp_body_aliased
        aliases = {11: 0}
    return pl.pallas_call(
        body,
        grid=grid,
        in_specs=in_specs,
        out_specs=pl.BlockSpec((1, 1, BLK), lambda i, blk0=blk0: (i + blk0, 0, 0)),
        out_shape=jax.ShapeDtypeStruct((B // BLK, 1, BLK), jnp.float32),
        input_output_aliases=aliases,
    )(*args)


@jax.jit
def kernel(h, r, t, entity_table, relation_table, W1, b1, W2, b2, W3, b3):
    h2 = h.reshape(B // IDX_COLS, IDX_COLS)
    r2 = r.reshape(B // IDX_COLS, IDX_COLS)
    t2 = t.reshape(B // IDX_COLS, IDX_COLS)
    w1h = W1[0:D]
    w1r = W1[D:2 * D]
    w1t = W1[2 * D:3 * D]
    b1_2 = b1.reshape(1, H1)
    b2_2 = b2.reshape(1, H2)
    w3_row = W3.reshape(1, H2)

    o = None
    for p in range(NSPLIT):
        e = _gather_call(p)(h2, r2, t2, entity_table, relation_table)
        o = _mlp_part(p, *e, w1h, w1r, w1t, b1_2, W2, b2_2, w3_row, b3,
                      o_prev=o)
    return o.reshape(B, 1)


# speedup vs baseline: 1.1812x; 1.0183x over previous
import functools

import jax
import jax.numpy as jnp
from jax import lax
from jax.experimental import pallas as pl
from jax.experimental.pallas import tpu as pltpu
from jax.experimental.pallas import tpu_sc as plsc

B = 16384
D = 128
H1 = 256
H2 = 128

NW = 32
CHUNK = 128
IDX_COLS = 128
NSPLIT = 2
HALF = B // NSPLIT


def _gather_body(part, h2, r2, t2, ent, rel, out_h, out_r, out_t,
                 hidx, ridx, tidx,
                 buf0, buf1, buf2, buf3, buf4, buf5,
                 gsem0, gsem1, gsem2, gsem3, gsem4, gsem5,
                 wsem0, wsem1, wsem2, wsem3, wsem4, wsem5):
    rows_per_w = HALF // NW
    chunks_per_w = max(1, rows_per_w // CHUNK)
    c = lax.axis_index("c")
    s = lax.axis_index("s")
    wid = s * 2 + c
    row0 = part * (HALF // IDX_COLS) + wid * chunks_per_w
    base = wid * rows_per_w

    pltpu.sync_copy(h2.at[pl.ds(row0, chunks_per_w)], hidx)
    pltpu.sync_copy(r2.at[pl.ds(row0, chunks_per_w)], ridx)
    pltpu.sync_copy(t2.at[pl.ds(row0, chunks_per_w)], tidx)

    jobs = []
    for tbl, idx, out in ((ent, hidx, out_h), (rel, ridx, out_r),
                          (ent, tidx, out_t)):
        for j in range(chunks_per_w):
            jobs.append((tbl, idx, j, out))
    n = len(jobs)
    bufs = (buf0, buf1, buf2, buf3, buf4, buf5)
    gsems = (gsem0, gsem1, gsem2, gsem3, gsem4, gsem5)
    wsems = (wsem0, wsem1, wsem2, wsem3, wsem4, wsem5)

    g = [pltpu.async_copy(jobs[k][0].at[jobs[k][1].at[jobs[k][2]]],
                          bufs[k], gsems[k]) for k in range(n)]
    wb = [None] * n
    for k in range(n):
        tbl, idx, j, out = jobs[k]
        g[k].wait()
        wb[k] = pltpu.async_copy(bufs[k],
                                 out.at[pl.ds(base + j * CHUNK, CHUNK)],
                                 wsems[k])
    for k in range(n):
        wb[k].wait()


@functools.cache
def _gather_call(part):
    return functools.partial(
        pl.kernel,
        mesh=plsc.VectorSubcoreMesh(core_axis_name="c", subcore_axis_name="s"),
        out_type=[
            jax.ShapeDtypeStruct((HALF, D), jnp.float32),
            jax.ShapeDtypeStruct((HALF, D), jnp.float32),
            jax.ShapeDtypeStruct((HALF, D), jnp.float32),
        ],
        scratch_types=[
            pltpu.VMEM((HALF // NW // CHUNK, IDX_COLS), jnp.int32),
            pltpu.VMEM((HALF // NW // CHUNK, IDX_COLS), jnp.int32),
            pltpu.VMEM((HALF // NW // CHUNK, IDX_COLS), jnp.int32),
            pltpu.VMEM((CHUNK, D), jnp.float32),
            pltpu.VMEM((CHUNK, D), jnp.float32),
            pltpu.VMEM((CHUNK, D), jnp.float32),
            pltpu.VMEM((CHUNK, D), jnp.float32),
            pltpu.VMEM((CHUNK, D), jnp.float32),
            pltpu.VMEM((CHUNK, D), jnp.float32),
            pltpu.SemaphoreType.DMA,
            pltpu.SemaphoreType.DMA,
            pltpu.SemaphoreType.DMA,
            pltpu.SemaphoreType.DMA,
            pltpu.SemaphoreType.DMA,
            pltpu.SemaphoreType.DMA,
            pltpu.SemaphoreType.DMA,
            pltpu.SemaphoreType.DMA,
            pltpu.SemaphoreType.DMA,
            pltpu.SemaphoreType.DMA,
            pltpu.SemaphoreType.DMA,
            pltpu.SemaphoreType.DMA,
        ],
    )(functools.partial(_gather_body, part))


BLK = 4096


def _mlp_body(h_ref, r_ref, t_ref, w1h, w1r, w1t, b1, w2, b2, w3r, b3, out_ref):
    x = (jnp.dot(h_ref[...], w1h[...], preferred_element_type=jnp.float32)
         + jnp.dot(r_ref[...], w1r[...], preferred_element_type=jnp.float32)
         + jnp.dot(t_ref[...], w1t[...], preferred_element_type=jnp.float32)
         + b1[...])
    x = jnp.maximum(x, 0.0)
    x = jnp.dot(x, w2[...], preferred_element_type=jnp.float32) + b2[...]
    x = jnp.maximum(x, 0.0)
    o = lax.dot_general(w3r[...], x, (((1,), (1,)), ((), ())),
                        preferred_element_type=jnp.float32) + b3[0]
    out_ref[...] = jax.nn.sigmoid(o).reshape(1, 1, BLK)


def _mlp_body_aliased(h_ref, r_ref, t_ref, w1h, w1r, w1t, b1, w2, b2, w3r, b3,
                      oprev, out_ref):
    del oprev
    _mlp_body(h_ref, r_ref, t_ref, w1h, w1r, w1t, b1, w2, b2, w3r, b3, out_ref)


def _mlp_part(part, h_e, r_e, t_e, w1h, w1r, w1t, b1_2, W2, b2_2, W3, b3,
              o_prev=None):
    grid = (HALF // BLK,)
    blk0 = part * (HALF // BLK)
    full = lambda i: (0, 0)
    in_specs = [
        pl.BlockSpec((BLK, D), lambda i: (i, 0)),
        pl.BlockSpec((BLK, D), lambda i: (i, 0)),
        pl.BlockSpec((BLK, D), lambda i: (i, 0)),
        pl.BlockSpec((D, H1), full),
        pl.BlockSpec((D, H1), full),
        pl.BlockSpec((D, H1), full),
        pl.BlockSpec((1, H1), full),
        pl.BlockSpec((H1, H2), full),
        pl.BlockSpec((1, H2), full),
        pl.BlockSpec((1, H2), full),
        pl.BlockSpec(memory_space=pltpu.SMEM),
    ]
    args = [h_e, r_e, t_e, w1h, w1r, w1t, b1_2, W2, b2_2, W3, b3]
    body = _mlp_body
    aliases = {}
    if o_prev is not None:
        in_specs.append(pl.BlockSpec(memory_space=pl.ANY))
        args.append(o_prev)
        body = _mlp_body_aliased
        aliases = {11: 0}
    return pl.pallas_call(
        body,
        grid=grid,
        in_specs=in_specs,
        out_specs=pl.BlockSpec((1, 1, BLK), lambda i, blk0=blk0: (i + blk0, 0, 0)),
        out_shape=jax.ShapeDtypeStruct((B // BLK, 1, BLK), jnp.float32),
        input_output_aliases=aliases,
    )(*args)


@jax.jit
def kernel(h, r, t, entity_table, relation_table, W1, b1, W2, b2, W3, b3):
    h2 = h.reshape(B // IDX_COLS, IDX_COLS)
    r2 = r.reshape(B // IDX_COLS, IDX_COLS)
    t2 = t.reshape(B // IDX_COLS, IDX_COLS)
    w1h = W1[0:D]
    w1r = W1[D:2 * D]
    w1t = W1[2 * D:3 * D]
    b1_2 = b1.reshape(1, H1)
    b2_2 = b2.reshape(1, H2)
    w3_row = W3.reshape(1, H2)

    o = None
    for p in range(NSPLIT):
        e = _gather_call(p)(h2, r2, t2, entity_table, relation_table)
        o = _mlp_part(p, *e, w1h, w1r, w1t, b1_2, W2, b2_2, w3_row, b3,
                      o_prev=o)
    return o.reshape(B, 1)


# speedup vs baseline: 1.2288x; 1.0403x over previous
import functools

import jax
import jax.numpy as jnp
from jax import lax
from jax.experimental import pallas as pl
from jax.experimental.pallas import tpu as pltpu
from jax.experimental.pallas import tpu_sc as plsc

B = 16384
D = 128
H1 = 256
H2 = 128

NW = 32
CHUNK = 128
IDX_COLS = 128
NSPLIT = 2
HALF = B // NSPLIT


def _gather_body(part, h1, r1, t1, ent, rel, out_h, out_r, out_t,
                 hidx, ridx, tidx,
                 buf0, buf1, buf2,
                 gsem0, gsem1, gsem2,
                 isem0, isem1, isem2,
                 wsem0, wsem1, wsem2):
    rows_per_w = HALF // NW
    c = lax.axis_index("c")
    s = lax.axis_index("s")
    wid = s * 2 + c
    flat0 = part * HALF + wid * rows_per_w
    base = wid * rows_per_w

    i0 = pltpu.async_copy(h1.at[pl.ds(flat0, rows_per_w)], hidx, isem0)
    i1 = pltpu.async_copy(r1.at[pl.ds(flat0, rows_per_w)], ridx, isem1)
    i2 = pltpu.async_copy(t1.at[pl.ds(flat0, rows_per_w)], tidx, isem2)

    jobs = ((ent, hidx, out_h, i0), (rel, ridx, out_r, i1),
            (ent, tidx, out_t, i2))
    bufs = (buf0, buf1, buf2)
    gsems = (gsem0, gsem1, gsem2)
    wsems = (wsem0, wsem1, wsem2)

    g = [None] * 3
    for k, (tbl, idx, out, ic) in enumerate(jobs):
        ic.wait()
        g[k] = pltpu.async_copy(tbl.at[idx], bufs[k], gsems[k])
    wb = [None] * 3
    for k, (tbl, idx, out, ic) in enumerate(jobs):
        g[k].wait()
        wb[k] = pltpu.async_copy(bufs[k], out.at[pl.ds(base, rows_per_w)],
                                 wsems[k])
    for k in range(3):
        wb[k].wait()


@functools.cache
def _gather_call(part):
    return functools.partial(
        pl.kernel,
        mesh=plsc.VectorSubcoreMesh(core_axis_name="c", subcore_axis_name="s"),
        out_type=[
            jax.ShapeDtypeStruct((HALF, D), jnp.float32),
            jax.ShapeDtypeStruct((HALF, D), jnp.float32),
            jax.ShapeDtypeStruct((HALF, D), jnp.float32),
        ],
        scratch_types=[
            pltpu.VMEM((HALF // NW,), jnp.int32),
            pltpu.VMEM((HALF // NW,), jnp.int32),
            pltpu.VMEM((HALF // NW,), jnp.int32),
            pltpu.VMEM((HALF // NW, D), jnp.float32),
            pltpu.VMEM((HALF // NW, D), jnp.float32),
            pltpu.VMEM((HALF // NW, D), jnp.float32),
            pltpu.SemaphoreType.DMA,
            pltpu.SemaphoreType.DMA,
            pltpu.SemaphoreType.DMA,
            pltpu.SemaphoreType.DMA,
            pltpu.SemaphoreType.DMA,
            pltpu.SemaphoreType.DMA,
            pltpu.SemaphoreType.DMA,
            pltpu.SemaphoreType.DMA,
            pltpu.SemaphoreType.DMA,
        ],
    )(functools.partial(_gather_body, part))


BLK = 4096


def _mlp_body(h_ref, r_ref, t_ref, w1h, w1r, w1t, b1, w2, b2, w3r, b3, out_ref):
    x = (jnp.dot(h_ref[...], w1h[...], preferred_element_type=jnp.float32)
         + jnp.dot(r_ref[...], w1r[...], preferred_element_type=jnp.float32)
         + jnp.dot(t_ref[...], w1t[...], preferred_element_type=jnp.float32)
         + b1[...])
    x = jnp.maximum(x, 0.0)
    x = jnp.dot(x, w2[...], preferred_element_type=jnp.float32) + b2[...]
    x = jnp.maximum(x, 0.0)
    o = lax.dot_general(w3r[...], x, (((1,), (1,)), ((), ())),
                        preferred_element_type=jnp.float32) + b3[0]
    out_ref[...] = jax.nn.sigmoid(o).reshape(1, 1, BLK)


def _mlp_body_aliased(h_ref, r_ref, t_ref, w1h, w1r, w1t, b1, w2, b2, w3r, b3,
                      oprev, out_ref):
    del oprev
    _mlp_body(h_ref, r_ref, t_ref, w1h, w1r, w1t, b1, w2, b2, w3r, b3, out_ref)


def _mlp_part(part, h_e, r_e, t_e, w1h, w1r, w1t, b1_2, W2, b2_2, W3, b3,
              o_prev=None):
    grid = (HALF // BLK,)
    blk0 = part * (HALF // BLK)
    full = lambda i: (0, 0)
    in_specs = [
        pl.BlockSpec((BLK, D), lambda i: (i, 0)),
        pl.BlockSpec((BLK, D), lambda i: (i, 0)),
        pl.BlockSpec((BLK, D), lambda i: (i, 0)),
        pl.BlockSpec((D, H1), full),
        pl.BlockSpec((D, H1), full),
        pl.BlockSpec((D, H1), full),
        pl.BlockSpec((1, H1), full),
        pl.BlockSpec((H1, H2), full),
        pl.BlockSpec((1, H2), full),
        pl.BlockSpec((1, H2), full),
        pl.BlockSpec(memory_space=pltpu.SMEM),
    ]
    args = [h_e, r_e, t_e, w1h, w1r, w1t, b1_2, W2, b2_2, W3, b3]
    body = _mlp_body
    aliases = {}
    if o_prev is not None:
        in_specs.append(pl.BlockSpec(memory_space=pl.ANY))
        args.append(o_prev)
        body = _mlp_body_aliased
        aliases = {11: 0}
    return pl.pallas_call(
        body,
        grid=grid,
        in_specs=in_specs,
        out_specs=pl.BlockSpec((1, 1, BLK), lambda i, blk0=blk0: (i + blk0, 0, 0)),
        out_shape=jax.ShapeDtypeStruct((B // BLK, 1, BLK), jnp.float32),
        input_output_aliases=aliases,
    )(*args)


@jax.jit
def kernel(h, r, t, entity_table, relation_table, W1, b1, W2, b2, W3, b3):
    w1h = W1[0:D]
    w1r = W1[D:2 * D]
    w1t = W1[2 * D:3 * D]
    b1_2 = b1.reshape(1, H1)
    b2_2 = b2.reshape(1, H2)
    w3_row = W3.reshape(1, H2)

    o = None
    for p in range(NSPLIT):
        e = _gather_call(p)(h, r, t, entity_table, relation_table)
        o = _mlp_part(p, *e, w1h, w1r, w1t, b1_2, W2, b2_2, w3_row, b3,
                      o_prev=o)
    return o.reshape(B, 1)
